# baseline (speedup 1.0000x reference)
import functools

import jax
import jax.numpy as jnp
from jax import lax
from jax.experimental import pallas as pl
from jax.experimental.pallas import tpu as pltpu
from jax.experimental.pallas import tpu_sc as plsc

T = 2048
D = 768
E = 8
H = 2048

BM = 256
NB = 24
NP = NB * BM
NW = 32
PW = 2 * T // NW
RW = NP // NW
CH = RW // 2
TW = T // NW


def _make_noise_const():
    import numpy as np
    with jax.default_device(jax.local_devices(backend="cpu")[0]):
        return np.asarray(
            jax.random.normal(jax.random.key(42), (1, T, E), jnp.float32))[0]


_NOISE = _make_noise_const()


def _router_body(x_ref, wr_ref, wn_ref, br_ref, bn_ref, noise_ref,
                 pos_ref, g_ref, bexp_ref, oh_ref):
    x = x_ref[...]
    wrn = jnp.concatenate([wr_ref[...], wn_ref[...]], axis=1)
    brn = jnp.concatenate([br_ref[...], bn_ref[...]], axis=1)
    lg = jnp.dot(x, wrn, preferred_element_type=jnp.float32) + brn
    logits = lg[:, :E]
    nlog = lg[:, E:]
    sp = jnp.maximum(nlog, 0.0) + jnp.log1p(jnp.exp(-jnp.abs(nlog)))
    noisy = logits + noise_ref[...] * sp

    lanes = lax.broadcasted_iota(jnp.int32, (T, E), 1)
    m1 = jnp.max(noisy, axis=1, keepdims=True)
    i1 = jnp.min(jnp.where(noisy == m1, lanes, E), axis=1, keepdims=True)
    masked = jnp.where(lanes == i1, -jnp.inf, noisy)
    m2 = jnp.max(masked, axis=1, keepdims=True)
    i2 = jnp.min(jnp.where(masked == m2, lanes, E), axis=1, keepdims=True)
    z = jnp.exp(m2 - m1)
    g1 = 1.0 / (1.0 + z)
    g2 = 1.0 - g1
    g_ref[...] = jnp.concatenate([g1, g2], axis=1)

    oh = jnp.where(lanes == i1, 1.0, 0.0) + jnp.where(lanes == i2, 1.0, 0.0)
    oh_ref[...] = oh

    rr = lax.broadcasted_iota(jnp.int32, (128, 128), 0)
    cc = lax.broadcasted_iota(jnp.int32, (128, 128), 1)
    ltri = jnp.where(rr > cc, 1.0, 0.0)

    def step(i, carry):
        blk = oh_ref[pl.ds(i * 128, 128), :]
        part = jnp.dot(ltri, blk, preferred_element_type=jnp.float32) + carry
        oh_ref[pl.ds(i * 128, 128), :] = part
        return carry + jnp.sum(blk, axis=0, keepdims=True)

    counts = lax.fori_loop(0, T // 128, step, jnp.zeros((1, E), jnp.float32))

    aligned = jnp.ceil(counts / BM) * BM
    r8 = lax.broadcasted_iota(jnp.int32, (E, E), 0)
    c8 = lax.broadcasted_iota(jnp.int32, (E, E), 1)
    stri = jnp.where(r8 < c8, 1.0, 0.0)
    starts = jnp.dot(aligned, stri, preferred_element_type=jnp.float32)

    excl = oh_ref[...]
    s_bc = jnp.broadcast_to(starts, (T, E))
    p1 = jnp.sum(jnp.where(lanes == i1, s_bc + excl, 0.0), axis=1, keepdims=True)
    p2 = jnp.sum(jnp.where(lanes == i2, s_bc + excl, 0.0), axis=1, keepdims=True)
    pos_ref[...] = jnp.concatenate([p1, p2], axis=1).astype(jnp.int32)

    bi = lax.broadcasted_iota(jnp.int32, (NB, E), 0).astype(jnp.float32) * float(BM)
    sb = jnp.broadcast_to(starts, (NB, E))
    bexp_ref[...] = (jnp.sum(jnp.where(sb <= bi, 1.0, 0.0), axis=1, keepdims=True)
                     - 1.0).astype(jnp.int32)


@functools.cache
def _sc_mesh():
    return plsc.VectorSubcoreMesh(
        core_axis_name="c", subcore_axis_name="s", num_cores=2, num_subcores=16)


def _wid():
    return lax.axis_index("s") * 2 + lax.axis_index("c")


def _dispatch_body(pos_hbm, g_hbm, x_hbm, xs_hbm, rg_hbm,
                   pos0_v, pos1_v, g0_v, g1_v, xr0_v, xr1_v,
                   semx, semy, sem1, sem2, sem3, sem4):
    base = _wid() * PW
    t0 = pl.multiple_of(base & (T - 1), PW)
    hp = PW // 2
    cpx = pltpu.async_copy(x_hbm.at[pl.ds(t0, hp)], xr0_v, semx)
    cpy = pltpu.async_copy(x_hbm.at[pl.ds(t0 + hp, hp)], xr1_v, semy)
    pltpu.sync_copy(pos_hbm.at[pl.ds(base, hp)], pos0_v)
    pltpu.sync_copy(pos_hbm.at[pl.ds(base + hp, hp)], pos1_v)
    pltpu.sync_copy(g_hbm.at[pl.ds(base, hp)], g0_v)
    pltpu.sync_copy(g_hbm.at[pl.ds(base + hp, hp)], g1_v)
    sg0 = pltpu.async_copy(g0_v, rg_hbm.at[pos0_v], sem3)
    sg1 = pltpu.async_copy(g1_v, rg_hbm.at[pos1_v], sem4)
    cpx.wait()
    s0 = pltpu.async_copy(xr0_v, xs_hbm.at[pos0_v], sem1)
    cpy.wait()
    s1 = pltpu.async_copy(xr1_v, xs_hbm.at[pos1_v], sem2)
    s0.wait()
    s1.wait()
    sg0.wait()
    sg1.wait()


def _combine_body(pos_hbm, y_hbm, out_hbm, pos_v, rows0_v, rows1_v, out_v, sem, sem2):
    t0 = _wid() * TW
    pltpu.sync_copy(pos_hbm.at[pl.ds(t0, TW)], pos_v.at[pl.ds(0, TW)])
    pltpu.sync_copy(pos_hbm.at[pl.ds(T + t0, TW)], pos_v.at[pl.ds(TW, TW)])
    hw = TW // 2
    for half in range(2):
        c0 = pltpu.async_copy(y_hbm.at[pos_v.at[pl.ds(half * hw, hw)]], rows0_v, sem)
        c1 = pltpu.async_copy(y_hbm.at[pos_v.at[pl.ds(TW + half * hw, hw)]], rows1_v, sem2)
        c0.wait()
        c1.wait()

        def tok(i, _):
            for c in range(D // 16):
                a = rows0_v[i, pl.ds(c * 16, 16)]
                b = rows1_v[i, pl.ds(c * 16, 16)]
                out_v[i, pl.ds(c * 16, 16)] = a + b
            return 0

        lax.fori_loop(0, hw, tok, 0)
        pltpu.sync_copy(out_v, out_hbm.at[pl.ds(t0 + half * hw, hw)])


def _mm_body(bexp_ref, xs_ref, rg_ref, w1_ref, w3_ref, w2_ref, y_ref):
    xb = xs_ref[...].astype(jnp.bfloat16)
    h1 = jnp.dot(xb, w1_ref[0].astype(jnp.bfloat16), preferred_element_type=jnp.float32)
    h3 = jnp.dot(xb, w3_ref[0].astype(jnp.bfloat16), preferred_element_type=jnp.float32)
    h = ((h1 * lax.logistic(h1)) * (h3 * rg_ref[...])).astype(jnp.bfloat16)
    y_ref[...] = jnp.dot(h, w2_ref[0].astype(jnp.bfloat16),
                         preferred_element_type=jnp.float32)


@jax.jit
def kernel(x, Wr, br, Wn, bn, w1, w2, w3):
    xf = x.reshape(T, D)
    noise = jnp.asarray(_NOISE)

    pos, g, bexp = pl.pallas_call(
        _router_body,
        out_shape=(
            jax.ShapeDtypeStruct((T, 2), jnp.int32),
            jax.ShapeDtypeStruct((T, 2), jnp.float32),
            jax.ShapeDtypeStruct((NB, 1), jnp.int32),
        ),
        scratch_shapes=[pltpu.VMEM((T, E), jnp.float32)],
    )(xf, Wr, Wn, br.reshape(1, E), bn.reshape(1, E), noise)

    pos_flat = pos.T.reshape(2 * T)
    g_flat = g.T.reshape(2 * T)
    bexp_flat = bexp.reshape(NB)

    dispatch = pl.kernel(
        _dispatch_body,
        out_type=(
            jax.ShapeDtypeStruct((NP, D), jnp.float32),
            jax.ShapeDtypeStruct((NP,), jnp.float32),
        ),
        mesh=_sc_mesh(),
        scratch_types=[
            pltpu.VMEM((PW // 2,), jnp.int32),
            pltpu.VMEM((PW // 2,), jnp.int32),
            pltpu.VMEM((PW // 2,), jnp.float32),
            pltpu.VMEM((PW // 2,), jnp.float32),
            pltpu.VMEM((PW // 2, D), jnp.float32),
            pltpu.VMEM((PW // 2, D), jnp.float32),
            pltpu.SemaphoreType.DMA,
            pltpu.SemaphoreType.DMA,
            pltpu.SemaphoreType.DMA,
            pltpu.SemaphoreType.DMA,
            pltpu.SemaphoreType.DMA,
            pltpu.SemaphoreType.DMA,
        ],
    )
    xs, row_gate = dispatch(pos_flat, g_flat, xf)

    y = pl.pallas_call(
        _mm_body,
        grid_spec=pltpu.PrefetchScalarGridSpec(
            num_scalar_prefetch=1,
            grid=(NB,),
            in_specs=[
                pl.BlockSpec((BM, D), lambda b, be: (b, 0)),
                pl.BlockSpec((BM, 1), lambda b, be: (b, 0)),
                pl.BlockSpec((1, D, H), lambda b, be: (be[b], 0, 0)),
                pl.BlockSpec((1, D, H), lambda b, be: (be[b], 0, 0)),
                pl.BlockSpec((1, H, D), lambda b, be: (be[b], 0, 0)),
            ],
            out_specs=pl.BlockSpec((BM, D), lambda b, be: (b, 0)),
        ),
        out_shape=jax.ShapeDtypeStruct((NP, D), jnp.float32),
    )(bexp_flat, xs, row_gate.reshape(NP, 1), w1, w3, w2)

    combine = pl.kernel(
        _combine_body,
        out_type=jax.ShapeDtypeStruct((T, D), jnp.float32),
        mesh=_sc_mesh(),
        scratch_types=[
            pltpu.VMEM((2 * TW,), jnp.int32),
            pltpu.VMEM((TW // 2, D), jnp.float32),
            pltpu.VMEM((TW // 2, D), jnp.float32),
            pltpu.VMEM((TW // 2, D), jnp.float32),
            pltpu.SemaphoreType.DMA,
            pltpu.SemaphoreType.DMA,
        ],
    )
    out = combine(pos_flat, y)

    return out.reshape(1, T, D)


# speedup vs baseline: 1.0260x; 1.0260x over previous
import functools

import jax
import jax.numpy as jnp
from jax import lax
from jax.experimental import pallas as pl
from jax.experimental.pallas import tpu as pltpu
from jax.experimental.pallas import tpu_sc as plsc

T = 2048
D = 768
E = 8
H = 2048

BM = 256
NB = 24
NP = NB * BM
NW = 32
PW = 2 * T // NW
RW = NP // NW
CH = RW // 2
TW = T // NW


def _make_noise_const():
    import numpy as np
    with jax.default_device(jax.local_devices(backend="cpu")[0]):
        return np.asarray(
            jax.random.normal(jax.random.key(42), (1, T, E), jnp.float32))[0]


_NOISE = _make_noise_const()


def _router_body(x_ref, wr_ref, wn_ref, br_ref, bn_ref, noise_ref,
                 pos_ref, g_ref, bexp_ref, oh_ref):
    x = x_ref[...]
    wrn = jnp.concatenate([wr_ref[...], wn_ref[...]], axis=1)
    brn = jnp.concatenate([br_ref[...], bn_ref[...]], axis=1)
    lg = jnp.dot(x, wrn, preferred_element_type=jnp.float32) + brn
    logits = lg[:, :E]
    nlog = lg[:, E:]
    sp = jnp.maximum(nlog, 0.0) + jnp.log1p(jnp.exp(-jnp.abs(nlog)))
    noisy = logits + noise_ref[...] * sp

    lanes = lax.broadcasted_iota(jnp.int32, (T, E), 1)
    m1 = jnp.max(noisy, axis=1, keepdims=True)
    i1 = jnp.min(jnp.where(noisy == m1, lanes, E), axis=1, keepdims=True)
    masked = jnp.where(lanes == i1, -jnp.inf, noisy)
    m2 = jnp.max(masked, axis=1, keepdims=True)
    i2 = jnp.min(jnp.where(masked == m2, lanes, E), axis=1, keepdims=True)
    z = jnp.exp(m2 - m1)
    g1 = 1.0 / (1.0 + z)
    g2 = 1.0 - g1
    g_ref[...] = jnp.concatenate([g1, g2], axis=1)

    oh = jnp.where(lanes == i1, 1.0, 0.0) + jnp.where(lanes == i2, 1.0, 0.0)
    oh_ref[...] = oh

    rr = lax.broadcasted_iota(jnp.int32, (128, 128), 0)
    cc = lax.broadcasted_iota(jnp.int32, (128, 128), 1)
    ltri = jnp.where(rr > cc, 1.0, 0.0)

    def step(i, carry):
        blk = oh_ref[pl.ds(i * 128, 128), :]
        part = jnp.dot(ltri, blk, preferred_element_type=jnp.float32) + carry
        oh_ref[pl.ds(i * 128, 128), :] = part
        return carry + jnp.sum(blk, axis=0, keepdims=True)

    counts = lax.fori_loop(0, T // 128, step, jnp.zeros((1, E), jnp.float32))

    aligned = jnp.ceil(counts / BM) * BM
    r8 = lax.broadcasted_iota(jnp.int32, (E, E), 0)
    c8 = lax.broadcasted_iota(jnp.int32, (E, E), 1)
    stri = jnp.where(r8 < c8, 1.0, 0.0)
    starts = jnp.dot(aligned, stri, preferred_element_type=jnp.float32)

    excl = oh_ref[...]
    s_bc = jnp.broadcast_to(starts, (T, E))
    p1 = jnp.sum(jnp.where(lanes == i1, s_bc + excl, 0.0), axis=1, keepdims=True)
    p2 = jnp.sum(jnp.where(lanes == i2, s_bc + excl, 0.0), axis=1, keepdims=True)
    pos_ref[...] = jnp.concatenate([p1, p2], axis=1).astype(jnp.int32)

    bi = lax.broadcasted_iota(jnp.int32, (NB, E), 0).astype(jnp.float32) * float(BM)
    sb = jnp.broadcast_to(starts, (NB, E))
    bexp_ref[...] = (jnp.sum(jnp.where(sb <= bi, 1.0, 0.0), axis=1, keepdims=True)
                     - 1.0).astype(jnp.int32)


@functools.cache
def _sc_mesh():
    return plsc.VectorSubcoreMesh(
        core_axis_name="c", subcore_axis_name="s", num_cores=2, num_subcores=16)


def _wid():
    return lax.axis_index("s") * 2 + lax.axis_index("c")


def _dispatch_body(pos_hbm, g_hbm, x_hbm, xs_hbm, rg_hbm,
                   pos_v, g_v, xrows_v, semx, sem1, sem2):
    base = _wid() * PW
    t0 = pl.multiple_of(base & (T - 1), PW)
    cpx = pltpu.async_copy(x_hbm.at[pl.ds(t0, PW)], xrows_v, semx)
    pltpu.sync_copy(pos_hbm.at[pl.ds(base, PW)], pos_v)
    pltpu.sync_copy(g_hbm.at[pl.ds(base, PW)], g_v)
    s2 = pltpu.async_copy(g_v, rg_hbm.at[pos_v], sem2)
    cpx.wait()
    pltpu.async_copy(xrows_v, xs_hbm.at[pos_v], sem1).wait()
    s2.wait()


def _combine_body(pos_hbm, y_hbm, out_hbm, pos_v, rows0_v, rows1_v, out_v, sem, sem2):
    t0 = _wid() * TW
    pltpu.sync_copy(pos_hbm.at[pl.ds(t0, TW)], pos_v.at[pl.ds(0, TW)])
    pltpu.sync_copy(pos_hbm.at[pl.ds(T + t0, TW)], pos_v.at[pl.ds(TW, TW)])
    hw = TW // 2
    for half in range(2):
        c0 = pltpu.async_copy(y_hbm.at[pos_v.at[pl.ds(half * hw, hw)]], rows0_v, sem)
        c1 = pltpu.async_copy(y_hbm.at[pos_v.at[pl.ds(TW + half * hw, hw)]], rows1_v, sem2)
        c0.wait()
        c1.wait()

        def tok(i, _):
            for c in range(D // 16):
                a = rows0_v[i, pl.ds(c * 16, 16)]
                b = rows1_v[i, pl.ds(c * 16, 16)]
                out_v[i, pl.ds(c * 16, 16)] = a + b
            return 0

        lax.fori_loop(0, hw, tok, 0)
        pltpu.sync_copy(out_v, out_hbm.at[pl.ds(t0 + half * hw, hw)])


def _mm_body(bexp_ref, xs_ref, rg_ref, w1_ref, w3_ref, w2_ref, y_ref):
    xb = xs_ref[...].astype(jnp.bfloat16)
    h1 = jnp.dot(xb, w1_ref[0].astype(jnp.bfloat16), preferred_element_type=jnp.float32)
    h3 = jnp.dot(xb, w3_ref[0].astype(jnp.bfloat16), preferred_element_type=jnp.float32)
    h = ((h1 * lax.logistic(h1)) * (h3 * rg_ref[...])).astype(jnp.bfloat16)
    y_ref[...] = jnp.dot(h, w2_ref[0].astype(jnp.bfloat16),
                         preferred_element_type=jnp.float32)


@jax.jit
def kernel(x, Wr, br, Wn, bn, w1, w2, w3):
    xf = x.reshape(T, D)
    noise = jnp.asarray(_NOISE)

    pos, g, bexp = pl.pallas_call(
        _router_body,
        out_shape=(
            jax.ShapeDtypeStruct((T, 2), jnp.int32),
            jax.ShapeDtypeStruct((T, 2), jnp.float32),
            jax.ShapeDtypeStruct((NB, 1), jnp.int32),
        ),
        scratch_shapes=[pltpu.VMEM((T, E), jnp.float32)],
    )(xf, Wr, Wn, br.reshape(1, E), bn.reshape(1, E), noise)

    pos_flat = pos.T.reshape(2 * T)
    g_flat = g.T.reshape(2 * T)
    bexp_flat = bexp.reshape(NB)

    dispatch = pl.kernel(
        _dispatch_body,
        out_type=(
            jax.ShapeDtypeStruct((NP, D), jnp.float32),
            jax.ShapeDtypeStruct((NP,), jnp.float32),
        ),
        mesh=_sc_mesh(),
        scratch_types=[
            pltpu.VMEM((PW,), jnp.int32),
            pltpu.VMEM((PW,), jnp.float32),
            pltpu.VMEM((PW, D), jnp.float32),
            pltpu.SemaphoreType.DMA,
            pltpu.SemaphoreType.DMA,
            pltpu.SemaphoreType.DMA,
        ],
    )
    xs, row_gate = dispatch(pos_flat, g_flat, xf)

    y = pl.pallas_call(
        _mm_body,
        grid_spec=pltpu.PrefetchScalarGridSpec(
            num_scalar_prefetch=1,
            grid=(NB,),
            in_specs=[
                pl.BlockSpec((BM, D), lambda b, be: (b, 0)),
                pl.BlockSpec((BM, 1), lambda b, be: (b, 0)),
                pl.BlockSpec((1, D, H), lambda b, be: (be[b], 0, 0)),
                pl.BlockSpec((1, D, H), lambda b, be: (be[b], 0, 0)),
                pl.BlockSpec((1, H, D), lambda b, be: (be[b], 0, 0)),
            ],
            out_specs=pl.BlockSpec((BM, D), lambda b, be: (b, 0)),
        ),
        out_shape=jax.ShapeDtypeStruct((NP, D), jnp.float32),
    )(bexp_flat, xs, row_gate.reshape(NP, 1), w1, w3, w2)

    combine = pl.kernel(
        _combine_body,
        out_type=jax.ShapeDtypeStruct((T, D), jnp.float32),
        mesh=_sc_mesh(),
        scratch_types=[
            pltpu.VMEM((2 * TW,), jnp.int32),
            pltpu.VMEM((TW // 2, D), jnp.float32),
            pltpu.VMEM((TW // 2, D), jnp.float32),
            pltpu.VMEM((TW // 2, D), jnp.float32),
            pltpu.SemaphoreType.DMA,
            pltpu.SemaphoreType.DMA,
        ],
    )
    out = combine(pos_flat, y)

    return out.reshape(1, T, D)


# speedup vs baseline: 1.0307x; 1.0046x over previous
import functools

import jax
import jax.numpy as jnp
from jax import lax
from jax.experimental import pallas as pl
from jax.experimental.pallas import tpu as pltpu
from jax.experimental.pallas import tpu_sc as plsc

T = 2048
D = 768
E = 8
H = 2048

BM = 256
NB = 24
NP = NB * BM
NW = 32
PW = 2 * T // NW
RW = NP // NW
CH = RW // 2
TW = T // NW


def _make_noise_const():
    import numpy as np
    with jax.default_device(jax.local_devices(backend="cpu")[0]):
        return np.asarray(
            jax.random.normal(jax.random.key(42), (1, T, E), jnp.float32))[0]


_NOISE = _make_noise_const()


def _router_body(x_ref, wr_ref, wn_ref, br_ref, bn_ref, noise_ref,
                 pos_ref, g_ref, bexp_ref, oh_ref):
    x = x_ref[...]
    wrn = jnp.concatenate([wr_ref[...], wn_ref[...]], axis=1)
    brn = jnp.concatenate([br_ref[...], bn_ref[...]], axis=1)
    lg = jnp.dot(x, wrn, preferred_element_type=jnp.float32) + brn
    logits = lg[:, :E]
    nlog = lg[:, E:]
    sp = jnp.maximum(nlog, 0.0) + jnp.log1p(jnp.exp(-jnp.abs(nlog)))
    noisy = logits + noise_ref[...] * sp

    lanes = lax.broadcasted_iota(jnp.int32, (T, E), 1)
    m1 = jnp.max(noisy, axis=1, keepdims=True)
    i1 = jnp.min(jnp.where(noisy == m1, lanes, E), axis=1, keepdims=True)
    masked = jnp.where(lanes == i1, -jnp.inf, noisy)
    m2 = jnp.max(masked, axis=1, keepdims=True)
    i2 = jnp.min(jnp.where(masked == m2, lanes, E), axis=1, keepdims=True)
    z = jnp.exp(m2 - m1)
    g1 = 1.0 / (1.0 + z)
    g2 = 1.0 - g1
    g_ref[...] = jnp.concatenate([g1, g2], axis=1)

    oh = jnp.where(lanes == i1, 1.0, 0.0) + jnp.where(lanes == i2, 1.0, 0.0)
    oh_ref[...] = oh

    rr = lax.broadcasted_iota(jnp.int32, (128, 128), 0)
    cc = lax.broadcasted_iota(jnp.int32, (128, 128), 1)
    ltri = jnp.where(rr > cc, 1.0, 0.0)

    def step(i, carry):
        blk = oh_ref[pl.ds(i * 128, 128), :]
        part = jnp.dot(ltri, blk, preferred_element_type=jnp.float32) + carry
        oh_ref[pl.ds(i * 128, 128), :] = part
        return carry + jnp.sum(blk, axis=0, keepdims=True)

    counts = lax.fori_loop(0, T // 128, step, jnp.zeros((1, E), jnp.float32))

    aligned = jnp.ceil(counts / BM) * BM
    r8 = lax.broadcasted_iota(jnp.int32, (E, E), 0)
    c8 = lax.broadcasted_iota(jnp.int32, (E, E), 1)
    stri = jnp.where(r8 < c8, 1.0, 0.0)
    starts = jnp.dot(aligned, stri, preferred_element_type=jnp.float32)

    excl = oh_ref[...]
    s_bc = jnp.broadcast_to(starts, (T, E))
    p1 = jnp.sum(jnp.where(lanes == i1, s_bc + excl, 0.0), axis=1, keepdims=True)
    p2 = jnp.sum(jnp.where(lanes == i2, s_bc + excl, 0.0), axis=1, keepdims=True)
    pos_ref[...] = jnp.concatenate([p1, p2], axis=1).astype(jnp.int32)

    bi = lax.broadcasted_iota(jnp.int32, (NB, E), 0).astype(jnp.float32) * float(BM)
    sb = jnp.broadcast_to(starts, (NB, E))
    bexp_ref[...] = (jnp.sum(jnp.where(sb <= bi, 1.0, 0.0), axis=1, keepdims=True)
                     - 1.0).astype(jnp.int32)


@functools.cache
def _sc_mesh():
    return plsc.VectorSubcoreMesh(
        core_axis_name="c", subcore_axis_name="s", num_cores=2, num_subcores=16)


def _wid():
    return lax.axis_index("s") * 2 + lax.axis_index("c")


def _dispatch_body(pos_hbm, g_hbm, x_hbm, xs_hbm, rg_hbm,
                   pos_v, g_v, xrows_v, semx, sem1, sem2):
    base = _wid() * PW
    t0 = pl.multiple_of(base & (T - 1), PW)
    cpx = pltpu.async_copy(x_hbm.at[pl.ds(t0, PW)], xrows_v, semx)
    pltpu.sync_copy(pos_hbm.at[pl.ds(base, PW)], pos_v)
    pltpu.sync_copy(g_hbm.at[pl.ds(base, PW)], g_v)
    s2 = pltpu.async_copy(g_v, rg_hbm.at[pos_v], sem2)
    cpx.wait()
    pltpu.async_copy(xrows_v, xs_hbm.at[pos_v], sem1).wait()
    s2.wait()


def _combine_body(pos_hbm, y_hbm, out_hbm, pos_v,
                  rows0_v, rows1_v, rows0b_v, rows1b_v, out_v,
                  sem, sem2, semb, semb2):
    t0 = _wid() * TW
    pltpu.sync_copy(pos_hbm.at[pl.ds(t0, TW)], pos_v.at[pl.ds(0, TW)])
    pltpu.sync_copy(pos_hbm.at[pl.ds(T + t0, TW)], pos_v.at[pl.ds(TW, TW)])
    hw = TW // 2
    c0 = pltpu.async_copy(y_hbm.at[pos_v.at[pl.ds(0, hw)]], rows0_v, sem)
    c1 = pltpu.async_copy(y_hbm.at[pos_v.at[pl.ds(TW, hw)]], rows1_v, sem2)
    c0b = pltpu.async_copy(y_hbm.at[pos_v.at[pl.ds(hw, hw)]], rows0b_v, semb)
    c1b = pltpu.async_copy(y_hbm.at[pos_v.at[pl.ds(TW + hw, hw)]], rows1b_v, semb2)
    for half, (ca, cb, ra, rb) in enumerate(
            [(c0, c1, rows0_v, rows1_v), (c0b, c1b, rows0b_v, rows1b_v)]):
        ca.wait()
        cb.wait()

        def tok(i, _):
            for c in range(D // 16):
                a = ra[i, pl.ds(c * 16, 16)]
                b = rb[i, pl.ds(c * 16, 16)]
                out_v[i, pl.ds(c * 16, 16)] = a + b
            return 0

        lax.fori_loop(0, hw, tok, 0)
        pltpu.sync_copy(out_v, out_hbm.at[pl.ds(t0 + half * hw, hw)])


def _mm_body(bexp_ref, xs_ref, rg_ref, w1_ref, w3_ref, w2_ref, y_ref):
    xb = xs_ref[...].astype(jnp.bfloat16)
    h1 = jnp.dot(xb, w1_ref[0].astype(jnp.bfloat16), preferred_element_type=jnp.float32)
    h3 = jnp.dot(xb, w3_ref[0].astype(jnp.bfloat16), preferred_element_type=jnp.float32)
    h = ((h1 * lax.logistic(h1)) * (h3 * rg_ref[...])).astype(jnp.bfloat16)
    y_ref[...] = jnp.dot(h, w2_ref[0].astype(jnp.bfloat16),
                         preferred_element_type=jnp.float32)


@jax.jit
def kernel(x, Wr, br, Wn, bn, w1, w2, w3):
    xf = x.reshape(T, D)
    noise = jnp.asarray(_NOISE)

    pos, g, bexp = pl.pallas_call(
        _router_body,
        out_shape=(
            jax.ShapeDtypeStruct((T, 2), jnp.int32),
            jax.ShapeDtypeStruct((T, 2), jnp.float32),
            jax.ShapeDtypeStruct((NB, 1), jnp.int32),
        ),
        scratch_shapes=[pltpu.VMEM((T, E), jnp.float32)],
    )(xf, Wr, Wn, br.reshape(1, E), bn.reshape(1, E), noise)

    pos_flat = pos.T.reshape(2 * T)
    g_flat = g.T.reshape(2 * T)
    bexp_flat = bexp.reshape(NB)

    dispatch = pl.kernel(
        _dispatch_body,
        out_type=(
            jax.ShapeDtypeStruct((NP, D), jnp.float32),
            jax.ShapeDtypeStruct((NP,), jnp.float32),
        ),
        mesh=_sc_mesh(),
        scratch_types=[
            pltpu.VMEM((PW,), jnp.int32),
            pltpu.VMEM((PW,), jnp.float32),
            pltpu.VMEM((PW, D), jnp.float32),
            pltpu.SemaphoreType.DMA,
            pltpu.SemaphoreType.DMA,
            pltpu.SemaphoreType.DMA,
        ],
    )
    xs, row_gate = dispatch(pos_flat, g_flat, xf)

    y = pl.pallas_call(
        _mm_body,
        grid_spec=pltpu.PrefetchScalarGridSpec(
            num_scalar_prefetch=1,
            grid=(NB,),
            in_specs=[
                pl.BlockSpec((BM, D), lambda b, be: (b, 0)),
                pl.BlockSpec((BM, 1), lambda b, be: (b, 0)),
                pl.BlockSpec((1, D, H), lambda b, be: (be[b], 0, 0)),
                pl.BlockSpec((1, D, H), lambda b, be: (be[b], 0, 0)),
                pl.BlockSpec((1, H, D), lambda b, be: (be[b], 0, 0)),
            ],
            out_specs=pl.BlockSpec((BM, D), lambda b, be: (b, 0)),
        ),
        out_shape=jax.ShapeDtypeStruct((NP, D), jnp.float32),
    )(bexp_flat, xs, row_gate.reshape(NP, 1), w1, w3, w2)

    combine = pl.kernel(
        _combine_body,
        out_type=jax.ShapeDtypeStruct((T, D), jnp.float32),
        mesh=_sc_mesh(),
        scratch_types=[
            pltpu.VMEM((2 * TW,), jnp.int32),
            pltpu.VMEM((TW // 2, D), jnp.float32),
            pltpu.VMEM((TW // 2, D), jnp.float32),
            pltpu.VMEM((TW // 2, D), jnp.float32),
            pltpu.VMEM((TW // 2, D), jnp.float32),
            pltpu.VMEM((TW // 2, D), jnp.float32),
            pltpu.SemaphoreType.DMA,
            pltpu.SemaphoreType.DMA,
            pltpu.SemaphoreType.DMA,
            pltpu.SemaphoreType.DMA,
        ],
    )
    out = combine(pos_flat, y)

    return out.reshape(1, T, D)
